# Initial kernel scaffold; baseline (speedup 1.0000x reference)
import jax
import jax.numpy as jnp
from jax.experimental import pallas as pl


def kernel(mol_emb, mol_padding_mask, llm_mask, input_ids, W):
    raise NotImplementedError("write your pallas kernel here")


# speedup vs baseline: 1.3889x; 1.3889x over previous
import functools

import jax
import jax.numpy as jnp
from jax import lax
from jax.experimental import pallas as pl
from jax.experimental.pallas import tpu as pltpu
from jax.experimental.pallas import tpu_sc as plsc

_HID = 1024
_TOT = 4 * 4096

_info = plsc.get_sparse_core_info()
_NC, _NS = _info.num_cores, _info.num_subcores
_NW = _NC * _NS
_BPW = _TOT // _NW
_CHUNK = 32
_NCHUNK = _BPW // _CHUNK


def _embed_body(ids_hbm, table_hbm, out_hbm, idx_v, rows_v, gsem):
    wid = lax.axis_index("s") * _NC + lax.axis_index("c")
    base = wid * _BPW
    pltpu.sync_copy(ids_hbm.at[pl.ds(base, _BPW)], idx_v)
    for i in range(_BPW // 16):
        sl = pl.ds(i * 16, 16)
        idx_v[sl] = jnp.maximum(idx_v[sl], 0)
    for c in range(_NCHUNK):
        pltpu.async_copy(
            table_hbm.at[idx_v.at[pl.ds(c * _CHUNK, _CHUNK)]],
            rows_v, gsem).wait()
        pltpu.sync_copy(rows_v, out_hbm.at[pl.ds(base + c * _CHUNK, _CHUNK)])


_embed_gather = functools.partial(
    pl.kernel,
    mesh=plsc.VectorSubcoreMesh(core_axis_name="c", subcore_axis_name="s"),
    out_type=jax.ShapeDtypeStruct((_TOT, _HID), jnp.float32),
    scratch_types=[
        pltpu.VMEM((_BPW,), jnp.int32),
        pltpu.VMEM((_CHUNK, _HID), jnp.float32),
        pltpu.SemaphoreType.DMA,
    ],
)(_embed_body)


def kernel(mol_emb, mol_padding_mask, llm_mask, input_ids, W):
    flat_ids = input_ids.reshape(_TOT)
    text_embeds = _embed_gather(flat_ids, W)
    text_embeds = text_embeds.reshape(input_ids.shape + (_HID,))
    return (mol_emb, mol_padding_mask, text_embeds, llm_mask, input_ids)


# speedup vs baseline: 1.6058x; 1.1562x over previous
import functools

import jax
import jax.numpy as jnp
from jax import lax
from jax.experimental import pallas as pl
from jax.experimental.pallas import tpu as pltpu
from jax.experimental.pallas import tpu_sc as plsc

_HID = 1024
_TOT = 4 * 4096

_info = plsc.get_sparse_core_info()
_NC, _NS = _info.num_cores, _info.num_subcores
_NW = _NC * _NS
_BPW = _TOT // _NW
_CHUNK = 32
_NCHUNK = _BPW // _CHUNK


_NBUF = 3


def _embed_body(ids_hbm, table_hbm, out_hbm, idx_v, rows_v, gsem, osem):
    wid = lax.axis_index("s") * _NC + lax.axis_index("c")
    base = wid * _BPW
    pltpu.sync_copy(ids_hbm.at[pl.ds(base, _BPW)], idx_v)
    for i in range(_BPW // 16):
        sl = pl.ds(i * 16, 16)
        idx_v[sl] = jnp.maximum(idx_v[sl], 0)

    gaths = [None] * _NBUF
    outs = [None] * _NBUF

    def start_gather(c):
        b = c % _NBUF
        gaths[b] = pltpu.async_copy(
            table_hbm.at[idx_v.at[pl.ds(c * _CHUNK, _CHUNK)]],
            rows_v.at[b], gsem.at[b])

    def start_out(c):
        b = c % _NBUF
        outs[b] = pltpu.async_copy(
            rows_v.at[b], out_hbm.at[pl.ds(base + c * _CHUNK, _CHUNK)],
            osem.at[b])

    start_gather(0)
    start_gather(1)
    for c in range(_NCHUNK):
        b = c % _NBUF
        if c + 2 < _NCHUNK:
            if c >= 1:
                outs[(c + 2) % _NBUF].wait()
            start_gather(c + 2)
        gaths[b].wait()
        start_out(c)
    for c in range(_NCHUNK - _NBUF, _NCHUNK):
        outs[c % _NBUF].wait()


_embed_gather = functools.partial(
    pl.kernel,
    mesh=plsc.VectorSubcoreMesh(core_axis_name="c", subcore_axis_name="s"),
    out_type=jax.ShapeDtypeStruct((_TOT, _HID), jnp.float32),
    scratch_types=[
        pltpu.VMEM((_BPW,), jnp.int32),
        pltpu.VMEM((_NBUF, _CHUNK, _HID), jnp.float32),
        pltpu.SemaphoreType.DMA((_NBUF,)),
        pltpu.SemaphoreType.DMA((_NBUF,)),
    ],
)(_embed_body)


def kernel(mol_emb, mol_padding_mask, llm_mask, input_ids, W):
    flat_ids = input_ids.reshape(_TOT)
    text_embeds = _embed_gather(flat_ids, W)
    text_embeds = text_embeds.reshape(input_ids.shape + (_HID,))
    return (mol_emb, mol_padding_mask, text_embeds, llm_mask, input_ids)


# speedup vs baseline: 1.6137x; 1.0049x over previous
import functools

import jax
import jax.numpy as jnp
from jax import lax
from jax.experimental import pallas as pl
from jax.experimental.pallas import tpu as pltpu
from jax.experimental.pallas import tpu_sc as plsc

_HID = 1024
_TOT = 4 * 4096

_info = plsc.get_sparse_core_info()
_NC, _NS = _info.num_cores, _info.num_subcores
_NW = _NC * _NS
_BPW = _TOT // _NW
_CHUNK = 16
_NCHUNK = _BPW // _CHUNK


_NBUF = 6


def _embed_body(ids_hbm, table_hbm, out_hbm, idx_v, rows_v, gsem, osem):
    wid = lax.axis_index("s") * _NC + lax.axis_index("c")
    base = wid * _BPW
    pltpu.sync_copy(ids_hbm.at[pl.ds(base, _BPW)], idx_v)

    gaths = [None] * _NBUF
    outs = [None] * _NBUF

    def start_gather(c):
        b = c % _NBUF
        iv = jnp.maximum(idx_v[pl.ds(c * _CHUNK, _CHUNK)], 0)
        gaths[b] = pltpu.async_copy(table_hbm.at[iv], rows_v.at[b], gsem.at[b])

    def start_out(c):
        b = c % _NBUF
        outs[b] = pltpu.async_copy(
            rows_v.at[b], out_hbm.at[pl.ds(base + c * _CHUNK, _CHUNK)],
            osem.at[b])

    for c in range(_NBUF - 1):
        start_gather(c)
    for c in range(_NCHUNK):
        b = c % _NBUF
        a = c + _NBUF - 1
        if a < _NCHUNK:
            if c >= 1:
                outs[a % _NBUF].wait()
            start_gather(a)
        gaths[b].wait()
        start_out(c)
    for c in range(_NCHUNK - _NBUF, _NCHUNK):
        outs[c % _NBUF].wait()


_embed_gather = functools.partial(
    pl.kernel,
    mesh=plsc.VectorSubcoreMesh(core_axis_name="c", subcore_axis_name="s"),
    out_type=jax.ShapeDtypeStruct((_TOT, _HID), jnp.float32),
    scratch_types=[
        pltpu.VMEM((_BPW,), jnp.int32),
        pltpu.VMEM((_NBUF, _CHUNK, _HID), jnp.float32),
        pltpu.SemaphoreType.DMA((_NBUF,)),
        pltpu.SemaphoreType.DMA((_NBUF,)),
    ],
)(_embed_body)


def kernel(mol_emb, mol_padding_mask, llm_mask, input_ids, W):
    flat_ids = input_ids.reshape(_TOT)
    text_embeds = _embed_gather(flat_ids, W)
    text_embeds = text_embeds.reshape(input_ids.shape + (_HID,))
    return (mol_emb, mol_padding_mask, text_embeds, llm_mask, input_ids)


# speedup vs baseline: 1.6389x; 1.0156x over previous
import functools

import jax
import jax.numpy as jnp
from jax import lax
from jax.experimental import pallas as pl
from jax.experimental.pallas import tpu as pltpu
from jax.experimental.pallas import tpu_sc as plsc

_HID = 1024
_TOT = 4 * 4096

_info = plsc.get_sparse_core_info()
_NC, _NS = _info.num_cores, _info.num_subcores
_NW = _NC * _NS
_BPW = _TOT // _NW
_CHUNK = 16
_NCHUNK = _BPW // _CHUNK


_NBUF = 4
_NROUND = _NCHUNK // _NBUF


def _embed_body(ids_hbm, table_hbm, out_hbm, idx_v, rows_v, gsem, osem):
    wid = lax.axis_index("s") * _NC + lax.axis_index("c")
    base = wid * _BPW
    pltpu.sync_copy(ids_hbm.at[pl.ds(base, _BPW)], idx_v)

    def start_gather(c, b):
        iv = jnp.maximum(idx_v[pl.ds(c * _CHUNK, _CHUNK)], 0)
        pltpu.async_copy(table_hbm.at[iv], rows_v.at[b], gsem.at[b])

    def wait_gather(b):
        pltpu.make_async_copy(
            table_hbm.at[pl.ds(0, _CHUNK)], rows_v.at[b], gsem.at[b]).wait()

    def start_out(c, b):
        pltpu.async_copy(
            rows_v.at[b], out_hbm.at[pl.ds(base + c * _CHUNK, _CHUNK)],
            osem.at[b])

    def wait_out(b):
        pltpu.make_async_copy(
            rows_v.at[b], out_hbm.at[pl.ds(base, _CHUNK)], osem.at[b]).wait()

    for b in range(_NBUF):
        start_gather(b, b)

    def _round(g, carry):
        c0 = g * _NBUF
        for b in range(_NBUF):
            c = c0 + b
            wait_gather(b)
            start_out(c, b)
            wait_out(b)
            start_gather(c + _NBUF, b)
        return carry

    lax.fori_loop(0, _NROUND - 1, _round, 0)

    c0 = (_NROUND - 1) * _NBUF
    for b in range(_NBUF):
        wait_gather(b)
        start_out(c0 + b, b)
    for b in range(_NBUF):
        wait_out(b)


_embed_gather = functools.partial(
    pl.kernel,
    mesh=plsc.VectorSubcoreMesh(core_axis_name="c", subcore_axis_name="s"),
    out_type=jax.ShapeDtypeStruct((_TOT, _HID), jnp.float32),
    scratch_types=[
        pltpu.VMEM((_BPW,), jnp.int32),
        pltpu.VMEM((_NBUF, _CHUNK, _HID), jnp.float32),
        pltpu.SemaphoreType.DMA((_NBUF,)),
        pltpu.SemaphoreType.DMA((_NBUF,)),
    ],
)(_embed_body)


def kernel(mol_emb, mol_padding_mask, llm_mask, input_ids, W):
    flat_ids = input_ids.reshape(_TOT)
    text_embeds = _embed_gather(flat_ids, W)
    text_embeds = text_embeds.reshape(input_ids.shape + (_HID,))
    return (mol_emb, mol_padding_mask, text_embeds, llm_mask, input_ids)
